# baseline (speedup 1.0000x reference)
import functools

import jax
import jax.numpy as jnp
from jax import lax
from jax.experimental import pallas as pl
from jax.experimental.pallas import tpu as pltpu
from jax.experimental.pallas import tpu_sc as plsc

_B = 4096
_F = 26
_D = 16
_N = _B * _F
_FIELD_SIZE = 100000
_GPR = 128 // _D
_TBL_G = 2600000 // _GPR

_INFO = plsc.get_sparse_core_info()
_NW = _INFO.num_cores * _INFO.num_subcores
_PER_W = _N // _NW
_L = 16
_NCH = 13
_CH = _PER_W // _NCH
_OUT_ROWS_W = _PER_W * _D // 128


def _sc_lookup(x_flat, tbl8):
    mesh = plsc.VectorSubcoreMesh(core_axis_name="c", subcore_axis_name="s")

    @functools.partial(
        pl.kernel,
        mesh=mesh,
        out_type=jax.ShapeDtypeStruct((_N * _D // 128, 128), jnp.float32),
        scratch_types=[
            pltpu.VMEM((_PER_W,), jnp.int32),
            pltpu.VMEM((_PER_W,), jnp.int32),
            pltpu.VMEM((_PER_W,), jnp.int32),
            pltpu.VMEM((_CH, 128), jnp.float32),
            pltpu.VMEM((_OUT_ROWS_W, 128), jnp.float32),
            pltpu.SemaphoreType.DMA,
        ],
        compiler_params=pltpu.CompilerParams(needs_layout_passes=False),
    )
    def k(x_hbm, tbl_hbm, out_hbm, xv, gidx, addr, grp, outv, sem):
        wid = lax.axis_index("s") * _INFO.num_cores + lax.axis_index("c")
        base = wid * _PER_W
        pltpu.sync_copy(x_hbm.at[pl.ds(base, _PER_W)], xv)

        iota = lax.iota(jnp.int32, _L)

        def prep(i, _):
            pos = base + i * _L + iota
            idv = xv[pl.ds(i * _L, _L)] + lax.rem(pos, _F) * _FIELD_SIZE
            gidx[pl.ds(i * _L, _L)] = idv >> 3
            jl = lax.rem(i * _L, _CH) + iota
            addr[pl.ds(i * _L, _L)] = (jl << 7) + ((idv & 7) << 4)
            return 0

        lax.fori_loop(0, _PER_W // _L, prep, 0, unroll=4)

        for c in range(_NCH):
            pltpu.async_copy(tbl_hbm.at[gidx.at[pl.ds(c * _CH, _CH)]], grp, sem).wait()

            def block(b, _, c=c):
                start = c * _CH + b * _L
                av = addr[pl.ds(start, _L)]
                rows = av >> 7
                cols0 = av & 127
                jv = start + iota
                orow = jv >> 3
                ocol0 = (jv & 7) << 4
                for cc in range(_D):
                    vals = plsc.load_gather(grp, [rows, cols0 + cc])
                    plsc.store_scatter(outv, [orow, ocol0 + cc], vals)
                return 0

            lax.fori_loop(0, _CH // _L, block, 0)
        pltpu.sync_copy(outv, out_hbm.at[pl.ds(wid * _OUT_ROWS_W, _OUT_ROWS_W)])

    return k(x_flat, tbl8)


def kernel(x, table):
    x_flat = x.reshape(_N).astype(jnp.int32)
    tbl8 = table.reshape(_TBL_G, 128)
    out = _sc_lookup(x_flat, tbl8)
    return out.reshape(_B, _F, _D)


# speedup vs baseline: 3.3449x; 3.3449x over previous
import functools

import jax
import jax.numpy as jnp
from jax import lax
from jax.experimental import pallas as pl
from jax.experimental.pallas import tpu as pltpu
from jax.experimental.pallas import tpu_sc as plsc

_B = 4096
_F = 26
_D = 16
_FIELD = 100000
_WMAX = 51200
_WMAIN = 50176
_HCUT = 50000
_TSTART = 2598976
_TW = 1024
_L = 16
_BH = _B // 2
_ROWS = _F * _FIELD

_INFO = plsc.get_sparse_core_info()
_NC = _INFO.num_cores
_EH = _D // _NC


def _win_geom(f, hw):
    lo = f * _FIELD + hw * _HCUT
    wstart = (lo // 128) * 128
    wlen = _WMAIN
    if wstart + wlen > _ROWS:
        wlen = ((_ROWS - wstart) // 128) * 128
    return wstart, lo - wstart, wlen


def _sc_lookup(x_t, tbl_t, tail_t):
    mesh = plsc.VectorSubcoreMesh(core_axis_name="c", subcore_axis_name="s")

    @functools.partial(
        pl.kernel,
        mesh=mesh,
        out_type=jax.ShapeDtypeStruct((_F, _D, _B), jnp.float32),
        scratch_types=[
            pltpu.VMEM_SHARED((_EH, _WMAX), jnp.float32),
            pltpu.VMEM_SHARED((_EH, _B), jnp.float32),
            pltpu.VMEM((1, _WMAX), jnp.float32),
            pltpu.VMEM((8, _BH), jnp.int32),
            pltpu.VMEM((1, _BH), jnp.float32),
            pltpu.SemaphoreType.DMA,
            pltpu.SemaphoreType.DMA,
        ],
        compiler_params=pltpu.CompilerParams(needs_layout_passes=False),
    )
    def k(x_hbm, tbl_hbm, tail_hbm, out_hbm, shwin, shout, win, xblk, ocol, semw, semx):
        core = lax.axis_index("c")
        s = lax.axis_index("s")
        eloc = lax.rem(s, _EH)
        bh = s // _EH
        b0 = pl.multiple_of(bh * _BH, 128)
        e0 = pl.multiple_of(core * _EH, _EH)
        iota = lax.iota(jnp.int32, _L)
        zero = iota * 0

        for f in range(_F):
            if f % 8 == 0:
                pltpu.sync_copy(
                    x_hbm.at[pl.ds(f, 8), pl.ds(b0, _BH)],
                    xblk)

            for hw in range(2):
                wstart, adj, wlen = _win_geom(f, hw)
                base = f * _FIELD + hw * _HCUT

                @pl.when(s == 0)
                def _(wstart=wstart, wlen=wlen):
                    pltpu.sync_copy(
                        tbl_hbm.at[pl.ds(e0, _EH), pl.ds(wstart, wlen)],
                        shwin.at[:, pl.ds(0, wlen)])
                    if wlen < _WMAIN:
                        pltpu.sync_copy(
                            tail_hbm.at[pl.ds(e0, _EH), :],
                            shwin.at[:, pl.ds(wlen, _TW)])

                plsc.subcore_barrier()
                pltpu.sync_copy(shwin.at[pl.ds(eloc, 1), :], win)

                lo = hw * _HCUT
                tail_cut = wstart + wlen - f * _FIELD
                tail_adj = f * _FIELD - _TSTART + wlen

                def gather(i, _, f=f, hw=hw, adj=adj, lo=lo, wlen=wlen,
                           tail_cut=tail_cut, tail_adj=tail_adj):
                    idxv = xblk[f % 8, pl.ds(i * _L, _L)]
                    if wlen < _WMAIN:
                        pos = jnp.where(idxv < tail_cut, idxv + (adj - lo),
                                        idxv - (_TSTART - f * _FIELD) + wlen)
                    else:
                        pos = idxv + (adj - lo)
                    pos = jnp.clip(pos, 0, _WMAX - 1)
                    vals = plsc.load_gather(win, [zero, pos])
                    sl = pl.ds(i * _L, _L)
                    if hw == 0:
                        ocol[0, sl] = vals
                    else:
                        m = idxv >= _HCUT
                        ocol[0, sl] = jnp.where(m, vals, ocol[0, sl])
                    return 0

                lax.fori_loop(0, _BH // _L, gather, 0, unroll=4)

            pltpu.sync_copy(ocol, shout.at[pl.ds(eloc, 1), pl.ds(b0, _BH)])
            plsc.subcore_barrier()

            @pl.when(s == 0)
            def _():
                pltpu.sync_copy(shout, out_hbm.at[f, pl.ds(e0, _EH), :])

            plsc.subcore_barrier()

    return k(x_t, tbl_t, tail_t)


def kernel(x, table):
    x_t = jnp.pad(x.astype(jnp.int32), ((0, 0), (0, 32 - _F))).T
    tail_t = table[_TSTART:].T
    out = _sc_lookup(x_t, table.T, tail_t)
    return out.transpose(2, 0, 1)


# speedup vs baseline: 3.3832x; 1.0115x over previous
import functools

import jax
import jax.numpy as jnp
from jax import lax
from jax.experimental import pallas as pl
from jax.experimental.pallas import tpu as pltpu
from jax.experimental.pallas import tpu_sc as plsc

_B = 4096
_F = 26
_D = 16
_FIELD = 100000
_WMAX = 51200
_WMAIN = 50176
_HCUT = 50000
_TSTART = 2598976
_TW = 1024
_L = 16
_BH = _B // 2
_ROWS = _F * _FIELD

_INFO = plsc.get_sparse_core_info()
_NC = _INFO.num_cores
_EH = _D // _NC


def _win_geom(f, hw):
    lo = f * _FIELD + hw * _HCUT
    wstart = (lo // 128) * 128
    wlen = _WMAIN
    if wstart + wlen > _ROWS:
        wlen = ((_ROWS - wstart) // 128) * 128
    return wstart, lo - wstart, wlen


def _sc_lookup(x_t, tbl_t, tail_t):
    mesh = plsc.VectorSubcoreMesh(core_axis_name="c", subcore_axis_name="s")

    @functools.partial(
        pl.kernel,
        mesh=mesh,
        out_type=jax.ShapeDtypeStruct((_F, _D, _B), jnp.float32),
        scratch_types=[
            pltpu.VMEM_SHARED((_EH, _WMAX), jnp.float32),
            pltpu.VMEM_SHARED((2, _EH, _B), jnp.float32),
            pltpu.VMEM((1, _WMAX), jnp.float32),
            pltpu.VMEM((8, _BH), jnp.int32),
            pltpu.VMEM((1, _BH), jnp.float32),
            pltpu.SemaphoreType.DMA,
            pltpu.SemaphoreType.DMA,
        ],
        compiler_params=pltpu.CompilerParams(needs_layout_passes=False),
    )
    def k(x_hbm, tbl_hbm, tail_hbm, out_hbm, shwin, shout, win, xblk, ocol, semw, semx):
        core = lax.axis_index("c")
        s = lax.axis_index("s")
        eloc = lax.rem(s, _EH)
        bh = s // _EH
        b0 = pl.multiple_of(bh * _BH, 128)
        e0 = pl.multiple_of(core * _EH, _EH)
        iota = lax.iota(jnp.int32, _L)
        zero = iota * 0

        for f in range(_F):
            if f % 8 == 0:
                pltpu.sync_copy(
                    x_hbm.at[pl.ds(f, 8), pl.ds(b0, _BH)],
                    xblk)

            for hw in range(2):
                wstart, adj, wlen = _win_geom(f, hw)
                base = f * _FIELD + hw * _HCUT

                @pl.when(s == 0)
                def _(wstart=wstart, wlen=wlen):
                    pltpu.sync_copy(
                        tbl_hbm.at[pl.ds(e0, _EH), pl.ds(wstart, wlen)],
                        shwin.at[:, pl.ds(0, wlen)])
                    if wlen < _WMAIN:
                        pltpu.sync_copy(
                            tail_hbm.at[pl.ds(e0, _EH), :],
                            shwin.at[:, pl.ds(wlen, _TW)])

                plsc.subcore_barrier()
                pltpu.sync_copy(shwin.at[pl.ds(eloc, 1), :], win)

                lo = hw * _HCUT
                tail_cut = wstart + wlen - f * _FIELD
                tail_adj = f * _FIELD - _TSTART + wlen

                def gather(i, _, f=f, hw=hw, adj=adj, lo=lo, wlen=wlen,
                           tail_cut=tail_cut, tail_adj=tail_adj):
                    idxv = xblk[f % 8, pl.ds(i * _L, _L)]
                    if wlen < _WMAIN:
                        pos = jnp.where(idxv < tail_cut, idxv + (adj - lo),
                                        idxv - (_TSTART - f * _FIELD) + wlen)
                    else:
                        pos = idxv + (adj - lo)
                    pos = jnp.clip(pos, 0, _WMAX - 1)
                    vals = plsc.load_gather(win, [zero, pos])
                    sl = pl.ds(i * _L, _L)
                    if hw == 0:
                        ocol[0, sl] = vals
                    else:
                        m = idxv >= _HCUT
                        ocol[0, sl] = jnp.where(m, vals, ocol[0, sl])
                    return 0

                lax.fori_loop(0, _BH // _L, gather, 0, unroll=8)

            pltpu.sync_copy(ocol, shout.at[f % 2, pl.ds(eloc, 1), pl.ds(b0, _BH)])
            plsc.subcore_barrier()

            @pl.when(s == 0)
            def _():
                pltpu.sync_copy(shout.at[f % 2], out_hbm.at[f, pl.ds(e0, _EH), :])

    return k(x_t, tbl_t, tail_t)


def kernel(x, table):
    x_t = jnp.pad(x.astype(jnp.int32), ((0, 0), (0, 32 - _F))).T
    tail_t = table[_TSTART:].T
    out = _sc_lookup(x_t, table.T, tail_t)
    return out.transpose(2, 0, 1)
